# Initial kernel scaffold; baseline (speedup 1.0000x reference)
import jax
import jax.numpy as jnp
from jax.experimental import pallas as pl


def kernel(data, hidden, W_feat, al_feat, ar_feat, b_feat, W_time, al_time, ar_time, b_time, W_ih, W_hh, b_ih, b_hh):
    raise NotImplementedError("write your pallas kernel here")


# speedup vs baseline: 1.9288x; 1.9288x over previous
import functools

import jax
import jax.numpy as jnp
from jax.experimental import pallas as pl
from jax.experimental.pallas import tpu as pltpu

F = 64
N = F + 1
HID = 4 * F
KIN = N * F * 3
OUT_SIZE = F * F


def _gat_body(hF_ref, hT_ref, WfT_ref, WtT_ref, alF_ref, arF_ref, bF_ref,
              alT_ref, arT_ref, bT_ref, outF_ref, outT_ref):
    def one(h, WT, al, ar, b):
        feat = jnp.dot(h, WT, preferred_element_type=jnp.float32)
        el = feat * al
        er0 = feat[0:1, :] * ar
        e = el + er0
        e = jnp.where(e >= 0.0, e, 0.2 * e)
        m = jnp.max(e, axis=0, keepdims=True)
        w = jnp.exp(e - m)
        s = jnp.sum(w, axis=0, keepdims=True)
        att = jnp.sum(w * feat, axis=0, keepdims=True) / s
        return jnp.concatenate([att, feat[1:, :]], axis=0) + b

    outF_ref[...] = one(hF_ref[...], WfT_ref[...], alF_ref[...], arF_ref[...], bF_ref[...])
    outT_ref[...] = one(hT_ref[...], WtT_ref[...], alT_ref[...], arT_ref[...], bT_ref[...])


def _gru_body(x_ref, Wih_ref, Whh_ref, bih_ref, bhh_ref, h0_ref,
              out_ref, h1_ref, y_acc, g_acc):
    i = pl.program_id(0)
    x = x_ref[...]
    w = Wih_ref[...]
    y = jnp.sum(w * x[None, :], axis=1)
    y_acc[i, :] = y + bih_ref[i, :]
    h0 = h0_ref[...]
    gh = jnp.sum(Whh_ref[0] * h0[None, :], axis=1)
    g_acc[i, :] = gh + bhh_ref[i, :]

    @pl.when(i == 2)
    def _():
        r = jax.nn.sigmoid(y_acc[0, :] + g_acc[0, :])
        z = jax.nn.sigmoid(y_acc[1, :] + g_acc[1, :])
        n = jnp.tanh(y_acc[2, :] + r * g_acc[2, :])
        h1 = (1.0 - z) * n + z * h0
        out_ref[...] = jnp.concatenate(
            [h1, jnp.zeros((OUT_SIZE - HID,), jnp.float32)], axis=0)
        h1_ref[0, 0, :] = h1


def kernel(data, hidden, W_feat, al_feat, ar_feat, b_feat,
           W_time, al_time, ar_time, b_time, W_ih, W_hh, b_ih, b_hh):
    f32 = jnp.float32
    z1 = jnp.zeros((1, F), f32)
    hF = jnp.concatenate([z1, data], axis=0)
    hT = jnp.concatenate([z1, data.T], axis=0)

    gat = pl.pallas_call(
        _gat_body,
        out_shape=(jax.ShapeDtypeStruct((N, F), f32),
                   jax.ShapeDtypeStruct((N, F), f32)),
    )
    fRF, fRT = gat(hF, hT, W_feat.T, W_time.T,
                   al_feat.reshape(1, F), ar_feat.reshape(1, F), b_feat.reshape(1, F),
                   al_time.reshape(1, F), ar_time.reshape(1, F), b_time.reshape(1, F))

    x = jnp.stack([hF, fRF, fRT], axis=-1).reshape(KIN)

    gru = pl.pallas_call(
        _gru_body,
        grid=(3,),
        in_specs=[
            pl.BlockSpec((KIN,), lambda i: (0,)),
            pl.BlockSpec((HID, KIN), lambda i: (i, 0)),
            pl.BlockSpec((1, HID, HID), lambda i: (i, 0, 0)),
            pl.BlockSpec((3, HID), lambda i: (0, 0)),
            pl.BlockSpec((3, HID), lambda i: (0, 0)),
            pl.BlockSpec((HID,), lambda i: (0,)),
        ],
        out_specs=(pl.BlockSpec((OUT_SIZE,), lambda i: (0,)),
                   pl.BlockSpec((1, 1, HID), lambda i: (0, 0, 0))),
        out_shape=(jax.ShapeDtypeStruct((OUT_SIZE,), f32),
                   jax.ShapeDtypeStruct((1, 1, HID), f32)),
        scratch_shapes=[pltpu.VMEM((3, HID), f32), pltpu.VMEM((3, HID), f32)],
    )
    out, h1 = gru(x, W_ih, W_hh.reshape(3, HID, HID), b_ih.reshape(3, HID),
                  b_hh.reshape(3, HID), hidden.reshape(HID))
    return out, h1


# speedup vs baseline: 2.0251x; 1.0499x over previous
import jax
import jax.numpy as jnp
from jax.experimental import pallas as pl
from jax.experimental.pallas import tpu as pltpu

F = 64
N = F + 1
HID = 4 * F
KIN = N * F * 3
OUT_SIZE = F * F
RB = 128
NSTEP = (3 * HID) // RB


def _gat_body(hF_ref, hT_ref, WfT_ref, WtT_ref, alF_ref, arF_ref, bF_ref,
              alT_ref, arT_ref, bT_ref, outF_ref, outT_ref):
    def one(h, WT, al, ar, b):
        feat = jnp.dot(h, WT, preferred_element_type=jnp.float32)
        el = feat * al
        er0 = feat[0:1, :] * ar
        e = el + er0
        e = jnp.where(e >= 0.0, e, 0.2 * e)
        m = jnp.max(e, axis=0, keepdims=True)
        w = jnp.exp(e - m)
        s = jnp.sum(w, axis=0, keepdims=True)
        att = jnp.sum(w * feat, axis=0, keepdims=True) / s
        return jnp.concatenate([att, feat[1:, :]], axis=0) + b

    outF_ref[...] = one(hF_ref[...], WfT_ref[...], alF_ref[...], arF_ref[...], bF_ref[...])
    outT_ref[...] = one(hT_ref[...], WtT_ref[...], alT_ref[...], arT_ref[...], bT_ref[...])


def _gru_body(x_ref, Wih_ref, Whh_ref, bih_ref, bhh_ref, h0_ref,
              out_ref, h1_ref, y_acc, g_acc):
    i = pl.program_id(0)
    x = x_ref[...]
    w = Wih_ref[...]
    y_acc[i, :] = jnp.sum(w * x, axis=1) + bih_ref[0, i, :]
    h0 = h0_ref[...]
    g_acc[i, :] = jnp.sum(Whh_ref[...] * h0, axis=1) + bhh_ref[0, i, :]

    @pl.when(i == NSTEP - 1)
    def _epilogue():
        xr = jnp.concatenate([y_acc[0, :], y_acc[1, :]])
        xz = jnp.concatenate([y_acc[2, :], y_acc[3, :]])
        xn = jnp.concatenate([y_acc[4, :], y_acc[5, :]])
        hr = jnp.concatenate([g_acc[0, :], g_acc[1, :]])
        hz = jnp.concatenate([g_acc[2, :], g_acc[3, :]])
        hn = jnp.concatenate([g_acc[4, :], g_acc[5, :]])
        r = jax.nn.sigmoid(xr + hr)
        z = jax.nn.sigmoid(xz + hz)
        n = jnp.tanh(xn + r * hn)
        h1 = (1.0 - z) * n + z * h0[0]
        out_ref[...] = jnp.concatenate(
            [h1, jnp.zeros((OUT_SIZE - HID,), jnp.float32)], axis=0)
        h1_ref[0, 0, :] = h1


def kernel(data, hidden, W_feat, al_feat, ar_feat, b_feat,
           W_time, al_time, ar_time, b_time, W_ih, W_hh, b_ih, b_hh):
    f32 = jnp.float32
    z1 = jnp.zeros((1, F), f32)
    hF = jnp.concatenate([z1, data], axis=0)
    hT = jnp.concatenate([z1, data.T], axis=0)

    gat = pl.pallas_call(
        _gat_body,
        out_shape=(jax.ShapeDtypeStruct((N, F), f32),
                   jax.ShapeDtypeStruct((N, F), f32)),
    )
    fRF, fRT = gat(hF, hT, W_feat.T, W_time.T,
                   al_feat.reshape(1, F), ar_feat.reshape(1, F), b_feat.reshape(1, F),
                   al_time.reshape(1, F), ar_time.reshape(1, F), b_time.reshape(1, F))

    x = jnp.stack([hF, fRF, fRT], axis=-1).reshape(1, KIN)

    full = lambda shape: pl.BlockSpec(shape, lambda i: tuple(0 for _ in shape))
    gru = pl.pallas_call(
        _gru_body,
        grid=(NSTEP,),
        in_specs=[
            full((1, KIN)),
            pl.BlockSpec((RB, KIN), lambda i: (i, 0)),
            pl.BlockSpec((RB, HID), lambda i: (i, 0)),
            full((1, NSTEP, RB)), full((1, NSTEP, RB)),
            full((1, HID)),
        ],
        out_specs=(full((OUT_SIZE,)), full((1, 1, HID))),
        out_shape=(jax.ShapeDtypeStruct((OUT_SIZE,), f32),
                   jax.ShapeDtypeStruct((1, 1, HID), f32)),
        scratch_shapes=[pltpu.VMEM((NSTEP, RB), f32),
                        pltpu.VMEM((NSTEP, RB), f32)],
    )
    out, h1 = gru(x, W_ih, W_hh, b_ih.reshape(1, NSTEP, RB),
                  b_hh.reshape(1, NSTEP, RB), hidden.reshape(1, HID))
    return out, h1
